# baseline (speedup 1.0000x reference)
import dataclasses
import functools
import math

import jax
import jax.numpy as jnp
from jax import lax
from jax.experimental import pallas as pl
from jax.experimental.pallas import tpu as pltpu
from jax.experimental.pallas import tpu_sc as plsc

_BATCH = 16384
_DIM = 64
_NCLS = 1000000
_NSAMP = 100
_NPAD = 128
_NW = 32
_LPW = _BATCH // _NW
_BGRAN = _NCLS // 16

_BLK = 2048
_GRID = _BATCH // _BLK


def _sc_gather(proj_bf, bgran, ids3d, samp3d):
    mesh = plsc.VectorSubcoreMesh(core_axis_name="c", subcore_axis_name="s")
    cp = pltpu.CompilerParams(use_tc_tiling_on_sc=False)
    if "needs_layout_passes" in pltpu.CompilerParams.__dataclass_fields__:
        cp = dataclasses.replace(cp, needs_layout_passes=False)

    @functools.partial(
        pl.kernel,
        mesh=mesh,
        compiler_params=cp,
        out_type=(
            jax.ShapeDtypeStruct((_BATCH, _DIM), jnp.bfloat16),
            jax.ShapeDtypeStruct((_NW, 1, _LPW), jnp.float32),
            jax.ShapeDtypeStruct((_NPAD, _DIM), jnp.bfloat16),
            jax.ShapeDtypeStruct((1, _NPAD), jnp.float32),
        ),
        scratch_types=[
            pltpu.VMEM((1, _LPW), jnp.int32),
            pltpu.VMEM((4, 128), jnp.int32),
            pltpu.VMEM((4, 128), jnp.int32),
            pltpu.VMEM((_LPW, _DIM), jnp.bfloat16),
            pltpu.VMEM((_LPW, 16), jnp.float32),
            pltpu.VMEM((1, _LPW), jnp.float32),
            pltpu.VMEM((1, _NPAD), jnp.int32),
            pltpu.VMEM((_NPAD, _DIM), jnp.bfloat16),
            pltpu.VMEM((_NPAD, 16), jnp.float32),
            pltpu.VMEM((1, _NPAD), jnp.float32),
            pltpu.SemaphoreType.DMA,
            pltpu.SemaphoreType.DMA,
        ],
    )
    def k(proj_hbm, bgran_hbm, ids_hbm, samp_hbm,
          rows_out, bias_out, ps_out, bs_out,
          ids_v, idxbuf, bidx, rowbuf, bgran_v, bl_v,
          samp_v, psbuf, bsg_v, bs_local, sem_g, sem_m):
        wid = lax.axis_index("s") * 2 + lax.axis_index("c")
        pltpu.sync_copy(ids_hbm.at[wid], ids_v)

        @pl.loop(0, _LPW // 16)
        def _mkidx(g):
            idv = ids_v[0, pl.ds(16 * g, 16)]
            row = g >> 3
            col = 16 * (g & 7)
            idxbuf[row, pl.ds(col, 16)] = idv
            bidx[row, pl.ds(col, 16)] = idv >> 4

        for c in range(_LPW // 128):
            pltpu.async_copy(proj_hbm.at[idxbuf.at[c]],
                             rowbuf.at[pl.ds(128 * c, 128)], sem_g)
            pltpu.async_copy(bgran_hbm.at[bidx.at[c]],
                             bgran_v.at[pl.ds(128 * c, 128)], sem_m)
        for c in range(_LPW // 128):
            pltpu.make_async_copy(proj_hbm.at[idxbuf.at[c]],
                                  rowbuf.at[pl.ds(128 * c, 128)],
                                  sem_g).wait()
            pltpu.make_async_copy(bgran_hbm.at[bidx.at[c]],
                                  bgran_v.at[pl.ds(128 * c, 128)],
                                  sem_m).wait()

        @pl.loop(0, _LPW // 16)
        def _bx(g):
            idv = ids_v[0, pl.ds(16 * g, 16)]
            rowv = 16 * g + lax.iota(jnp.int32, 16)
            bl_v[0, pl.ds(16 * g, 16)] = plsc.load_gather(
                bgran_v, [rowv, idv & 15])

        pltpu.sync_copy(rowbuf,
                        rows_out.at[pl.ds(
                            pl.multiple_of(wid * _LPW, _LPW), _LPW)])
        pltpu.sync_copy(bl_v, bias_out.at[wid])

        @pl.when(wid == _NW - 1)
        def _samples():
            pltpu.sync_copy(samp_hbm.at[0], samp_v)

            @pl.loop(0, _NPAD // 16)
            def _mksidx(g):
                sv = samp_v[0, pl.ds(16 * g, 16)]
                idxbuf[0, pl.ds(16 * g, 16)] = sv
                bidx[0, pl.ds(16 * g, 16)] = sv >> 4
            pltpu.async_copy(proj_hbm.at[idxbuf.at[0]], psbuf, sem_g)
            pltpu.async_copy(bgran_hbm.at[bidx.at[0]], bsg_v, sem_m)
            pltpu.make_async_copy(proj_hbm.at[idxbuf.at[0]], psbuf,
                                  sem_g).wait()
            pltpu.make_async_copy(bgran_hbm.at[bidx.at[0]], bsg_v,
                                  sem_m).wait()

            @pl.loop(0, _NPAD // 16)
            def _sx(g):
                sv = samp_v[0, pl.ds(16 * g, 16)]
                rowv = 16 * g + lax.iota(jnp.int32, 16)
                bs_local[0, pl.ds(16 * g, 16)] = plsc.load_gather(
                    bsg_v, [rowv, sv & 15])
            pltpu.sync_copy(psbuf, ps_out)
            pltpu.sync_copy(bs_local, bs_out)

    return k(proj_bf, bgran, ids3d, samp3d)


def _tc_loss_body(pred_ref, rows_ref, bias_ref, lab_ref, ps_ref, bs_ref,
                  samp_ref, out_ref):
    i = pl.program_id(0)

    @pl.when(i == 0)
    def _init():
        out_ref[...] = jnp.zeros((1, 1), jnp.float32)

    pred = pred_ref[...]
    rows = rows_ref[...].astype(jnp.float32)
    bias_l = bias_ref[...]
    lab = lab_ref[...]
    ps = ps_ref[...].astype(jnp.float32)
    bs_row = bs_ref[...]
    samp = samp_ref[...]

    ls = jnp.sum(pred * rows, axis=1, keepdims=True) + bias_l

    noise = lax.dot_general(pred, ps, (((1,), (1,)), ((), ())),
                            preferred_element_type=jnp.float32)
    noise = noise + bs_row
    rej = (lab == samp)
    nrej = jnp.sum(rej.astype(jnp.float32), axis=1, keepdims=True)
    noise = noise - 1e6 * rej.astype(jnp.float32)
    noise = noise - jnp.log(float(_NSAMP) - nrej)
    col = lax.broadcasted_iota(jnp.int32, (_BLK, _NPAD), 1)
    noise = jnp.where(col < _NSAMP, noise, -1e30)

    m = jnp.maximum(ls, jnp.max(noise, axis=1, keepdims=True))
    se = jnp.exp(ls - m) + jnp.sum(jnp.exp(noise - m), axis=1, keepdims=True)
    nll = m + jnp.log(se) - ls
    out_ref[...] += jnp.sum(nll, axis=(0, 1), keepdims=True)

    @pl.when(i == _GRID - 1)
    def _fin():
        out_ref[...] = out_ref[...] / float(_BATCH)


def kernel(predictions, labels, projection, bias, samples):
    proj_bf = projection.astype(jnp.bfloat16)
    ids3d = labels.reshape(_NW, 1, _LPW)
    samp_gather = jnp.concatenate(
        [samples, jnp.zeros((_NPAD - _NSAMP,), jnp.int32)]).reshape(1, 1, _NPAD)

    rows, bias3, ps, bs = _sc_gather(proj_bf, bias.reshape(_BGRAN, 16),
                                     ids3d, samp_gather)

    bs_row = bs + math.log(_NCLS - 1)
    samp2d = jnp.concatenate(
        [samples, jnp.full((_NPAD - _NSAMP,), -1, jnp.int32)]).reshape(1, _NPAD)
    lab2d = labels.reshape(_BATCH, 1)
    bias2 = bias3.reshape(_BATCH, 1)

    out = pl.pallas_call(
        _tc_loss_body,
        grid=(_GRID,),
        in_specs=[
            pl.BlockSpec((_BLK, _DIM), lambda i: (i, 0)),
            pl.BlockSpec((_BLK, _DIM), lambda i: (i, 0)),
            pl.BlockSpec((_BLK, 1), lambda i: (i, 0)),
            pl.BlockSpec((_BLK, 1), lambda i: (i, 0)),
            pl.BlockSpec((_NPAD, _DIM), lambda i: (0, 0)),
            pl.BlockSpec((1, _NPAD), lambda i: (0, 0)),
            pl.BlockSpec((1, _NPAD), lambda i: (0, 0)),
        ],
        out_specs=pl.BlockSpec((1, 1), lambda i: (0, 0)),
        out_shape=jax.ShapeDtypeStruct((1, 1), jnp.float32),
    )(predictions, rows, bias2, lab2d, ps, bs_row, samp2d)
    return out[0, 0]


# speedup vs baseline: 1.3322x; 1.3322x over previous
import dataclasses
import functools
import math

import jax
import jax.numpy as jnp
from jax import lax
from jax.experimental import pallas as pl
from jax.experimental.pallas import tpu as pltpu
from jax.experimental.pallas import tpu_sc as plsc

_BATCH = 16384
_DIM = 64
_NCLS = 1000000
_NSAMP = 100
_NPAD = 128
_NW = 32
_LPW = _BATCH // _NW
_BGRAN = _NCLS // 16

_BLK = 2048
_GRID = _BATCH // _BLK


def _sc_gather(proj_bf, bgran, ids3d, samp3d):
    mesh = plsc.VectorSubcoreMesh(core_axis_name="c", subcore_axis_name="s")
    cp = pltpu.CompilerParams(use_tc_tiling_on_sc=False)
    if "needs_layout_passes" in pltpu.CompilerParams.__dataclass_fields__:
        cp = dataclasses.replace(cp, needs_layout_passes=False)

    @functools.partial(
        pl.kernel,
        mesh=mesh,
        compiler_params=cp,
        out_type=(
            jax.ShapeDtypeStruct((_BATCH, _DIM), jnp.float32),
            jax.ShapeDtypeStruct((_NW, 1, _LPW), jnp.float32),
            jax.ShapeDtypeStruct((_NPAD, _DIM), jnp.float32),
            jax.ShapeDtypeStruct((1, _NPAD), jnp.float32),
        ),
        scratch_types=[
            pltpu.VMEM((1, _LPW), jnp.int32),
            pltpu.VMEM((4, 128), jnp.int32),
            pltpu.VMEM((4, 128), jnp.int32),
            pltpu.VMEM((_LPW, _DIM), jnp.float32),
            pltpu.VMEM((_LPW, 16), jnp.float32),
            pltpu.VMEM((1, _LPW), jnp.float32),
            pltpu.VMEM((1, _NPAD), jnp.int32),
            pltpu.VMEM((_NPAD, _DIM), jnp.float32),
            pltpu.VMEM((_NPAD, 16), jnp.float32),
            pltpu.VMEM((1, _NPAD), jnp.float32),
            pltpu.SemaphoreType.DMA,
            pltpu.SemaphoreType.DMA,
        ],
    )
    def k(proj_hbm, bgran_hbm, ids_hbm, samp_hbm,
          rows_out, bias_out, ps_out, bs_out,
          ids_v, idxbuf, bidx, rowbuf, bgran_v, bl_v,
          samp_v, psbuf, bsg_v, bs_local, sem_g, sem_m):
        wid = lax.axis_index("s") * 2 + lax.axis_index("c")
        pltpu.sync_copy(ids_hbm.at[wid], ids_v)

        @pl.loop(0, _LPW // 16)
        def _mkidx(g):
            idv = ids_v[0, pl.ds(16 * g, 16)]
            row = g >> 3
            col = 16 * (g & 7)
            idxbuf[row, pl.ds(col, 16)] = idv
            bidx[row, pl.ds(col, 16)] = idv >> 4

        for c in range(_LPW // 128):
            pltpu.async_copy(proj_hbm.at[idxbuf.at[c]],
                             rowbuf.at[pl.ds(128 * c, 128)], sem_g)
            pltpu.async_copy(bgran_hbm.at[bidx.at[c]],
                             bgran_v.at[pl.ds(128 * c, 128)], sem_m)
        for c in range(_LPW // 128):
            pltpu.make_async_copy(proj_hbm.at[idxbuf.at[c]],
                                  rowbuf.at[pl.ds(128 * c, 128)],
                                  sem_g).wait()
            pltpu.make_async_copy(bgran_hbm.at[bidx.at[c]],
                                  bgran_v.at[pl.ds(128 * c, 128)],
                                  sem_m).wait()

        @pl.loop(0, _LPW // 16)
        def _bx(g):
            idv = ids_v[0, pl.ds(16 * g, 16)]
            rowv = 16 * g + lax.iota(jnp.int32, 16)
            bl_v[0, pl.ds(16 * g, 16)] = plsc.load_gather(
                bgran_v, [rowv, idv & 15])

        pltpu.sync_copy(rowbuf,
                        rows_out.at[pl.ds(
                            pl.multiple_of(wid * _LPW, _LPW), _LPW)])
        pltpu.sync_copy(bl_v, bias_out.at[wid])

        @pl.when(wid == _NW - 1)
        def _samples():
            pltpu.sync_copy(samp_hbm.at[0], samp_v)

            @pl.loop(0, _NPAD // 16)
            def _mksidx(g):
                sv = samp_v[0, pl.ds(16 * g, 16)]
                idxbuf[0, pl.ds(16 * g, 16)] = sv
                bidx[0, pl.ds(16 * g, 16)] = sv >> 4
            pltpu.async_copy(proj_hbm.at[idxbuf.at[0]], psbuf, sem_g)
            pltpu.async_copy(bgran_hbm.at[bidx.at[0]], bsg_v, sem_m)
            pltpu.make_async_copy(proj_hbm.at[idxbuf.at[0]], psbuf,
                                  sem_g).wait()
            pltpu.make_async_copy(bgran_hbm.at[bidx.at[0]], bsg_v,
                                  sem_m).wait()

            @pl.loop(0, _NPAD // 16)
            def _sx(g):
                sv = samp_v[0, pl.ds(16 * g, 16)]
                rowv = 16 * g + lax.iota(jnp.int32, 16)
                bs_local[0, pl.ds(16 * g, 16)] = plsc.load_gather(
                    bsg_v, [rowv, sv & 15])
            pltpu.sync_copy(psbuf, ps_out)
            pltpu.sync_copy(bs_local, bs_out)

    return k(proj_bf, bgran, ids3d, samp3d)


def _tc_loss_body(pred_ref, rows_ref, bias_ref, lab_ref, ps_ref, bs_ref,
                  samp_ref, out_ref):
    i = pl.program_id(0)

    @pl.when(i == 0)
    def _init():
        out_ref[...] = jnp.zeros((1, 1), jnp.float32)

    pred = pred_ref[...]
    rows = rows_ref[...]
    bias_l = bias_ref[...]
    lab = lab_ref[...]
    ps = ps_ref[...]
    bs_row = bs_ref[...]
    samp = samp_ref[...]

    ls = jnp.sum(pred * rows, axis=1, keepdims=True) + bias_l

    noise = lax.dot_general(pred, ps, (((1,), (1,)), ((), ())),
                            preferred_element_type=jnp.float32)
    noise = noise + bs_row
    rej = (lab == samp)
    nrej = jnp.sum(rej.astype(jnp.float32), axis=1, keepdims=True)
    noise = noise - 1e6 * rej.astype(jnp.float32)
    noise = noise - jnp.log(float(_NSAMP) - nrej)
    col = lax.broadcasted_iota(jnp.int32, (_BLK, _NPAD), 1)
    noise = jnp.where(col < _NSAMP, noise, -1e30)

    m = jnp.maximum(ls, jnp.max(noise, axis=1, keepdims=True))
    se = jnp.exp(ls - m) + jnp.sum(jnp.exp(noise - m), axis=1, keepdims=True)
    nll = m + jnp.log(se) - ls
    out_ref[...] += jnp.sum(nll, axis=(0, 1), keepdims=True)

    @pl.when(i == _GRID - 1)
    def _fin():
        out_ref[...] = out_ref[...] / float(_BATCH)


def kernel(predictions, labels, projection, bias, samples):
    ids3d = labels.reshape(_NW, 1, _LPW)
    samp_gather = jnp.concatenate(
        [samples, jnp.zeros((_NPAD - _NSAMP,), jnp.int32)]).reshape(1, 1, _NPAD)

    rows, bias3, ps, bs = _sc_gather(projection, bias.reshape(_BGRAN, 16),
                                     ids3d, samp_gather)

    bs_row = bs + math.log(_NCLS - 1)
    samp2d = jnp.concatenate(
        [samples, jnp.full((_NPAD - _NSAMP,), -1, jnp.int32)]).reshape(1, _NPAD)
    lab2d = labels.reshape(_BATCH, 1)
    bias2 = bias3.reshape(_BATCH, 1)

    out = pl.pallas_call(
        _tc_loss_body,
        grid=(_GRID,),
        in_specs=[
            pl.BlockSpec((_BLK, _DIM), lambda i: (i, 0)),
            pl.BlockSpec((_BLK, _DIM), lambda i: (i, 0)),
            pl.BlockSpec((_BLK, 1), lambda i: (i, 0)),
            pl.BlockSpec((_BLK, 1), lambda i: (i, 0)),
            pl.BlockSpec((_NPAD, _DIM), lambda i: (0, 0)),
            pl.BlockSpec((1, _NPAD), lambda i: (0, 0)),
            pl.BlockSpec((1, _NPAD), lambda i: (0, 0)),
        ],
        out_specs=pl.BlockSpec((1, 1), lambda i: (0, 0)),
        out_shape=jax.ShapeDtypeStruct((1, 1), jnp.float32),
    )(predictions, rows, bias2, lab2d, ps, bs_row, samp2d)
    return out[0, 0]


# speedup vs baseline: 1.6430x; 1.2333x over previous
import dataclasses
import functools
import math

import jax
import jax.numpy as jnp
from jax import lax
from jax.experimental import pallas as pl
from jax.experimental.pallas import tpu as pltpu
from jax.experimental.pallas import tpu_sc as plsc

_BATCH = 16384
_DIM = 64
_NCLS = 1000000
_NSAMP = 100
_NPAD = 128
_NW = 32
_LPW = _BATCH // _NW
_BPAD = 1000448
_BGRAN = _BPAD // 128

_TW = 1024
_TGRID = 489
_PROWS = _TGRID * _TW

_BLK = 2048
_GRID = _BATCH // _BLK


def _tr_body(slab_a_ref, slab_b_ref, eye_ref, out_ref):
    eye = eye_ref[...]
    ta = lax.dot_general(slab_a_ref[...], eye, (((0,), (0,)), ((), ())),
                         preferred_element_type=jnp.float32)
    tb = lax.dot_general(slab_b_ref[...], eye, (((0,), (0,)), ((), ())),
                         preferred_element_type=jnp.float32)
    out_ref[:, :_DIM] = ta
    out_ref[:, _DIM:] = tb


def _relayout(proj_t, eye):
    return pl.pallas_call(
        _tr_body,
        grid=(_TGRID,),
        in_specs=[
            pl.BlockSpec((_DIM, _TW), lambda k: (0, k)),
            pl.BlockSpec((_DIM, _TW),
                         lambda k: (0, jnp.minimum(k + _TGRID,
                                                   _NCLS // _TW))),
            pl.BlockSpec((_DIM, _DIM), lambda k: (0, 0)),
        ],
        out_specs=pl.BlockSpec((_TW, 128), lambda k: (k, 0)),
        out_shape=jax.ShapeDtypeStruct((_PROWS, 128), jnp.float32),
    )(proj_t, proj_t, eye)


def _sc_gather(proj2, bgran, ids3d, samp3d):
    mesh = plsc.VectorSubcoreMesh(core_axis_name="c", subcore_axis_name="s")
    cp = pltpu.CompilerParams()
    if "needs_layout_passes" in pltpu.CompilerParams.__dataclass_fields__:
        cp = dataclasses.replace(cp, needs_layout_passes=False)

    @functools.partial(
        pl.kernel,
        mesh=mesh,
        compiler_params=cp,
        out_type=(
            jax.ShapeDtypeStruct((_BATCH, 128), jnp.float32),
            jax.ShapeDtypeStruct((_NW, 1, _LPW), jnp.float32),
            jax.ShapeDtypeStruct((_NPAD, 128), jnp.float32),
            jax.ShapeDtypeStruct((1, _NPAD), jnp.float32),
        ),
        scratch_types=[
            pltpu.VMEM((1, _LPW), jnp.int32),
            pltpu.VMEM((4, 128), jnp.int32),
            pltpu.VMEM((4, 128), jnp.int32),
            pltpu.VMEM((_LPW, 128), jnp.float32),
            pltpu.VMEM((128, 128), jnp.float32),
            pltpu.VMEM((1, _LPW), jnp.float32),
            pltpu.VMEM((1, _NPAD), jnp.int32),
            pltpu.VMEM((_NPAD, 128), jnp.float32),
            pltpu.VMEM((1, _NPAD), jnp.float32),
            pltpu.SemaphoreType.DMA,
            pltpu.SemaphoreType.DMA,
        ],
    )
    def k(proj_hbm, bgran_hbm, ids_hbm, samp_hbm,
          rows_out, bias_out, ps_out, bs_out,
          ids_v, idxbuf, bidx, rowbuf, bgran_v, bl_v,
          samp_v, psbuf, bs_local, sem_g, sem_m):
        wid = lax.axis_index("s") * 2 + lax.axis_index("c")
        pltpu.sync_copy(ids_hbm.at[wid], ids_v)

        @pl.loop(0, _LPW // 16)
        def _mkidx(g):
            idv = ids_v[0, pl.ds(16 * g, 16)]
            row = g >> 3
            col = 16 * (g & 7)
            hi = (idv >= _PROWS).astype(jnp.int32)
            idxbuf[row, pl.ds(col, 16)] = idv - hi * _PROWS
            bidx[row, pl.ds(col, 16)] = idv >> 7

        for c in range(_LPW // 128):
            pltpu.async_copy(proj_hbm.at[idxbuf.at[c]],
                             rowbuf.at[pl.ds(128 * c, 128)], sem_g)

        for c in range(_LPW // 128):
            pltpu.sync_copy(bgran_hbm.at[bidx.at[c]], bgran_v)

            @pl.loop(0, 8)
            def _bx(g):
                j0 = 128 * c + 16 * g
                idv = ids_v[0, pl.ds(j0, 16)]
                rowv = 16 * g + lax.iota(jnp.int32, 16)
                bl_v[0, pl.ds(j0, 16)] = plsc.load_gather(
                    bgran_v, [rowv, idv & 127])

        for c in range(_LPW // 128):
            pltpu.make_async_copy(proj_hbm.at[idxbuf.at[c]],
                                  rowbuf.at[pl.ds(128 * c, 128)],
                                  sem_g).wait()

        pltpu.sync_copy(rowbuf,
                        rows_out.at[pl.ds(
                            pl.multiple_of(wid * _LPW, _LPW), _LPW)])
        pltpu.sync_copy(bl_v, bias_out.at[wid])

        @pl.when(wid == _NW - 1)
        def _samples():
            pltpu.sync_copy(samp_hbm.at[0], samp_v)

            @pl.loop(0, _NPAD // 16)
            def _mksidx(g):
                sv = samp_v[0, pl.ds(16 * g, 16)]
                shi = (sv >= _PROWS).astype(jnp.int32)
                idxbuf[0, pl.ds(16 * g, 16)] = sv - shi * _PROWS
                bidx[0, pl.ds(16 * g, 16)] = sv >> 7
            pltpu.async_copy(proj_hbm.at[idxbuf.at[0]], psbuf, sem_g)
            pltpu.sync_copy(bgran_hbm.at[bidx.at[0]], bgran_v)

            @pl.loop(0, _NPAD // 16)
            def _sx(g):
                sv = samp_v[0, pl.ds(16 * g, 16)]
                rowv = 16 * g + lax.iota(jnp.int32, 16)
                bs_local[0, pl.ds(16 * g, 16)] = plsc.load_gather(
                    bgran_v, [rowv, sv & 127])
            pltpu.make_async_copy(proj_hbm.at[idxbuf.at[0]], psbuf,
                                  sem_g).wait()
            pltpu.sync_copy(psbuf, ps_out)
            pltpu.sync_copy(bs_local, bs_out)

    return k(proj2, bgran, ids3d, samp3d)


def _tc_loss_body(pred_ref, rows_ref, bias_ref, lab_ref, ps_ref, bs_ref,
                  samp_ref, spar_ref, out_ref):
    i = pl.program_id(0)

    @pl.when(i == 0)
    def _init():
        out_ref[...] = jnp.zeros((1, 1), jnp.float32)

    pred = pred_ref[...]
    rows2 = rows_ref[...]
    bias_l = bias_ref[...]
    lab = lab_ref[...]
    ps2 = ps_ref[...]
    bs_row = bs_ref[...]
    samp = samp_ref[...]
    spar = spar_ref[...]

    odd = lab >= _PROWS
    ls_e = jnp.sum(pred * rows2[:, :_DIM], axis=1, keepdims=True)
    ls_o = jnp.sum(pred * rows2[:, _DIM:], axis=1, keepdims=True)
    ls = jnp.where(odd, ls_o, ls_e) + bias_l

    ps = jnp.where(spar == 1, ps2[:, _DIM:], ps2[:, :_DIM])
    noise = lax.dot_general(pred, ps, (((1,), (1,)), ((), ())),
                            preferred_element_type=jnp.float32)
    noise = noise + bs_row
    rej = (lab == samp)
    nrej = jnp.sum(rej.astype(jnp.float32), axis=1, keepdims=True)
    noise = noise - 1e6 * rej.astype(jnp.float32)
    noise = noise - jnp.log(float(_NSAMP) - nrej)
    col = lax.broadcasted_iota(jnp.int32, (_BLK, _NPAD), 1)
    noise = jnp.where(col < _NSAMP, noise, -1e30)

    m = jnp.maximum(ls, jnp.max(noise, axis=1, keepdims=True))
    se = jnp.exp(ls - m) + jnp.sum(jnp.exp(noise - m), axis=1, keepdims=True)
    nll = m + jnp.log(se) - ls
    out_ref[...] += jnp.sum(nll, axis=(0, 1), keepdims=True)

    @pl.when(i == _GRID - 1)
    def _fin():
        out_ref[...] = out_ref[...] / float(_BATCH)


def kernel(predictions, labels, projection, bias, samples):
    proj_t = jnp.transpose(projection)
    eye = jnp.eye(_DIM, dtype=jnp.float32)
    proj2 = _relayout(proj_t, eye)

    ids3d = labels.reshape(_NW, 1, _LPW)
    samp_gather = jnp.concatenate(
        [samples, jnp.zeros((_NPAD - _NSAMP,), jnp.int32)]).reshape(1, 1, _NPAD)
    bias2d = jnp.pad(bias, (0, _BPAD - _NCLS)).reshape(_BGRAN, 128)

    rows, bias3, ps2, bs = _sc_gather(proj2, bias2d, ids3d, samp_gather)

    bs_row = bs + math.log(_NCLS - 1)
    samp2d = jnp.concatenate(
        [samples, jnp.full((_NPAD - _NSAMP,), -1, jnp.int32)]).reshape(1, _NPAD)
    spar = (jnp.concatenate(
        [samples, jnp.zeros((_NPAD - _NSAMP,), jnp.int32)])
        >= _PROWS).astype(jnp.int32).reshape(_NPAD, 1)
    lab2d = labels.reshape(_BATCH, 1)
    bias2 = bias3.reshape(_BATCH, 1)

    out = pl.pallas_call(
        _tc_loss_body,
        grid=(_GRID,),
        in_specs=[
            pl.BlockSpec((_BLK, _DIM), lambda i: (i, 0)),
            pl.BlockSpec((_BLK, 128), lambda i: (i, 0)),
            pl.BlockSpec((_BLK, 1), lambda i: (i, 0)),
            pl.BlockSpec((_BLK, 1), lambda i: (i, 0)),
            pl.BlockSpec((_NPAD, 128), lambda i: (0, 0)),
            pl.BlockSpec((1, _NPAD), lambda i: (0, 0)),
            pl.BlockSpec((1, _NPAD), lambda i: (0, 0)),
            pl.BlockSpec((_NPAD, 1), lambda i: (0, 0)),
        ],
        out_specs=pl.BlockSpec((1, 1), lambda i: (0, 0)),
        out_shape=jax.ShapeDtypeStruct((1, 1), jnp.float32),
    )(predictions, rows, bias2, lab2d, ps2, bs_row, samp2d, spar)
    return out[0, 0]


# speedup vs baseline: 2.3506x; 1.4307x over previous
import dataclasses
import functools
import math

import jax
import jax.numpy as jnp
from jax import lax
from jax.experimental import pallas as pl
from jax.experimental.pallas import tpu as pltpu
from jax.experimental.pallas import tpu_sc as plsc

_BATCH = 16384
_DIM = 64
_NCLS = 1000000
_NSAMP = 100
_NPAD = 128
_NW = 32
_LPW = _BATCH // _NW
_BPAD = 1000448
_BGRAN = _BPAD // 128

_TW = 2048
_TGRID = 245
_PROWS = _TGRID * _TW

_BLK = 2048
_GRID = _BATCH // _BLK


def _tr_body(slab_a_ref, slab_b_ref, eye_ref, out_ref):
    eye = eye_ref[...]
    sa = slab_a_ref[...].astype(jnp.bfloat16)
    sb = slab_b_ref[...].astype(jnp.bfloat16)
    ta = lax.dot_general(sa, eye, (((0,), (0,)), ((), ())),
                         preferred_element_type=jnp.float32)
    tb = lax.dot_general(sb, eye, (((0,), (0,)), ((), ())),
                         preferred_element_type=jnp.float32)
    out_ref[:, :_DIM] = ta
    out_ref[:, _DIM:] = tb


def _relayout(proj_t, eye):
    return pl.pallas_call(
        _tr_body,
        grid=(_TGRID,),
        in_specs=[
            pl.BlockSpec((_DIM, _TW), lambda k: (0, k)),
            pl.BlockSpec((_DIM, _TW),
                         lambda k: (0, jnp.minimum(k + _TGRID,
                                                   _NCLS // _TW))),
            pl.BlockSpec((_DIM, _DIM), lambda k: (0, 0)),
        ],
        out_specs=pl.BlockSpec((_TW, 128), lambda k: (k, 0)),
        out_shape=jax.ShapeDtypeStruct((_PROWS, 128), jnp.float32),
    )(proj_t, proj_t, eye)


def _sc_gather(proj2, bgran, ids3d, samp3d):
    mesh = plsc.VectorSubcoreMesh(core_axis_name="c", subcore_axis_name="s")
    cp = pltpu.CompilerParams()
    if "needs_layout_passes" in pltpu.CompilerParams.__dataclass_fields__:
        cp = dataclasses.replace(cp, needs_layout_passes=False)

    @functools.partial(
        pl.kernel,
        mesh=mesh,
        compiler_params=cp,
        out_type=(
            jax.ShapeDtypeStruct((_BATCH, 128), jnp.float32),
            jax.ShapeDtypeStruct((_NW, 1, _LPW), jnp.float32),
            jax.ShapeDtypeStruct((_NPAD, 128), jnp.float32),
            jax.ShapeDtypeStruct((1, _NPAD), jnp.float32),
        ),
        scratch_types=[
            pltpu.VMEM((1, _LPW), jnp.int32),
            pltpu.VMEM((4, 128), jnp.int32),
            pltpu.VMEM((4, 128), jnp.int32),
            pltpu.VMEM((_LPW, 128), jnp.float32),
            pltpu.VMEM((128, 128), jnp.float32),
            pltpu.VMEM((1, _LPW), jnp.float32),
            pltpu.VMEM((1, _NPAD), jnp.int32),
            pltpu.VMEM((_NPAD, 128), jnp.float32),
            pltpu.VMEM((1, _NPAD), jnp.float32),
            pltpu.SemaphoreType.DMA,
            pltpu.SemaphoreType.DMA,
        ],
    )
    def k(proj_hbm, bgran_hbm, ids_hbm, samp_hbm,
          rows_out, bias_out, ps_out, bs_out,
          ids_v, idxbuf, bidx, rowbuf, bgran_v, bl_v,
          samp_v, psbuf, bs_local, sem_g, sem_m):
        wid = lax.axis_index("s") * 2 + lax.axis_index("c")
        pltpu.sync_copy(ids_hbm.at[wid], ids_v)

        @pl.loop(0, _LPW // 16)
        def _mkidx(g):
            idv = ids_v[0, pl.ds(16 * g, 16)]
            row = g >> 3
            col = 16 * (g & 7)
            hi = (idv >= _PROWS).astype(jnp.int32)
            idxbuf[row, pl.ds(col, 16)] = idv - hi * _PROWS
            bidx[row, pl.ds(col, 16)] = idv >> 7

        for c in range(_LPW // 128):
            pltpu.async_copy(proj_hbm.at[idxbuf.at[c]],
                             rowbuf.at[pl.ds(128 * c, 128)], sem_g)

        for c in range(_LPW // 128):
            pltpu.sync_copy(bgran_hbm.at[bidx.at[c]], bgran_v)

            @pl.loop(0, 8)
            def _bx(g):
                j0 = 128 * c + 16 * g
                idv = ids_v[0, pl.ds(j0, 16)]
                rowv = 16 * g + lax.iota(jnp.int32, 16)
                bl_v[0, pl.ds(j0, 16)] = plsc.load_gather(
                    bgran_v, [rowv, idv & 127])

        for c in range(_LPW // 128):
            pltpu.make_async_copy(proj_hbm.at[idxbuf.at[c]],
                                  rowbuf.at[pl.ds(128 * c, 128)],
                                  sem_g).wait()

        pltpu.sync_copy(rowbuf,
                        rows_out.at[pl.ds(
                            pl.multiple_of(wid * _LPW, _LPW), _LPW)])
        pltpu.sync_copy(bl_v, bias_out.at[wid])

        @pl.when(wid == _NW - 1)
        def _samples():
            pltpu.sync_copy(samp_hbm.at[0], samp_v)

            @pl.loop(0, _NPAD // 16)
            def _mksidx(g):
                sv = samp_v[0, pl.ds(16 * g, 16)]
                shi = (sv >= _PROWS).astype(jnp.int32)
                idxbuf[0, pl.ds(16 * g, 16)] = sv - shi * _PROWS
                bidx[0, pl.ds(16 * g, 16)] = sv >> 7
            pltpu.async_copy(proj_hbm.at[idxbuf.at[0]], psbuf, sem_g)
            pltpu.sync_copy(bgran_hbm.at[bidx.at[0]], bgran_v)

            @pl.loop(0, _NPAD // 16)
            def _sx(g):
                sv = samp_v[0, pl.ds(16 * g, 16)]
                rowv = 16 * g + lax.iota(jnp.int32, 16)
                bs_local[0, pl.ds(16 * g, 16)] = plsc.load_gather(
                    bgran_v, [rowv, sv & 127])
            pltpu.make_async_copy(proj_hbm.at[idxbuf.at[0]], psbuf,
                                  sem_g).wait()
            pltpu.sync_copy(psbuf, ps_out)
            pltpu.sync_copy(bs_local, bs_out)

    return k(proj2, bgran, ids3d, samp3d)


def _tc_loss_body(pred_ref, rows_ref, bias_ref, lab_ref, ps_ref, bs_ref,
                  samp_ref, spar_ref, out_ref):
    i = pl.program_id(0)

    @pl.when(i == 0)
    def _init():
        out_ref[...] = jnp.zeros((1, 1), jnp.float32)

    pred = pred_ref[...]
    rows2 = rows_ref[...]
    bias_l = bias_ref[...]
    lab = lab_ref[...]
    ps2 = ps_ref[...]
    bs_row = bs_ref[...]
    samp = samp_ref[...]
    spar = spar_ref[...]

    odd = lab >= _PROWS
    ls_e = jnp.sum(pred * rows2[:, :_DIM], axis=1, keepdims=True)
    ls_o = jnp.sum(pred * rows2[:, _DIM:], axis=1, keepdims=True)
    ls = jnp.where(odd, ls_o, ls_e) + bias_l

    ps = jnp.where(spar == 1, ps2[:, _DIM:], ps2[:, :_DIM])
    noise = lax.dot_general(pred, ps, (((1,), (1,)), ((), ())),
                            preferred_element_type=jnp.float32)
    noise = noise + bs_row
    rej = (lab == samp)
    nrej = jnp.sum(rej.astype(jnp.float32), axis=1, keepdims=True)
    noise = noise - 1e6 * rej.astype(jnp.float32)
    noise = noise - jnp.log(float(_NSAMP) - nrej)
    col = lax.broadcasted_iota(jnp.int32, (_BLK, _NPAD), 1)
    noise = jnp.where(col < _NSAMP, noise, -1e30)

    m = jnp.maximum(ls, jnp.max(noise, axis=1, keepdims=True))
    se = jnp.exp(ls - m) + jnp.sum(jnp.exp(noise - m), axis=1, keepdims=True)
    nll = m + jnp.log(se) - ls
    out_ref[...] += jnp.sum(nll, axis=(0, 1), keepdims=True)

    @pl.when(i == _GRID - 1)
    def _fin():
        out_ref[...] = out_ref[...] / float(_BATCH)


def kernel(predictions, labels, projection, bias, samples):
    proj_t = jnp.transpose(projection)
    eye = jnp.eye(_DIM, dtype=jnp.bfloat16)
    proj2 = _relayout(proj_t, eye)

    ids3d = labels.reshape(_NW, 1, _LPW)
    samp_gather = jnp.concatenate(
        [samples, jnp.zeros((_NPAD - _NSAMP,), jnp.int32)]).reshape(1, 1, _NPAD)
    bias2d = jnp.pad(bias, (0, _BPAD - _NCLS)).reshape(_BGRAN, 128)

    rows, bias3, ps2, bs = _sc_gather(proj2, bias2d, ids3d, samp_gather)

    bs_row = bs + math.log(_NCLS - 1)
    samp2d = jnp.concatenate(
        [samples, jnp.full((_NPAD - _NSAMP,), -1, jnp.int32)]).reshape(1, _NPAD)
    spar = (jnp.concatenate(
        [samples, jnp.zeros((_NPAD - _NSAMP,), jnp.int32)])
        >= _PROWS).astype(jnp.int32).reshape(_NPAD, 1)
    lab2d = labels.reshape(_BATCH, 1)
    bias2 = bias3.reshape(_BATCH, 1)

    out = pl.pallas_call(
        _tc_loss_body,
        grid=(_GRID,),
        in_specs=[
            pl.BlockSpec((_BLK, _DIM), lambda i: (i, 0)),
            pl.BlockSpec((_BLK, 128), lambda i: (i, 0)),
            pl.BlockSpec((_BLK, 1), lambda i: (i, 0)),
            pl.BlockSpec((_BLK, 1), lambda i: (i, 0)),
            pl.BlockSpec((_NPAD, 128), lambda i: (0, 0)),
            pl.BlockSpec((1, _NPAD), lambda i: (0, 0)),
            pl.BlockSpec((1, _NPAD), lambda i: (0, 0)),
            pl.BlockSpec((_NPAD, 1), lambda i: (0, 0)),
        ],
        out_specs=pl.BlockSpec((1, 1), lambda i: (0, 0)),
        out_shape=jax.ShapeDtypeStruct((1, 1), jnp.float32),
    )(predictions, rows, bias2, lab2d, ps2, bs_row, samp2d, spar)
    return out[0, 0]


# speedup vs baseline: 2.9074x; 1.2369x over previous
import dataclasses
import functools
import math

import jax
import jax.numpy as jnp
from jax import lax
from jax.experimental import pallas as pl
from jax.experimental.pallas import tpu as pltpu
from jax.experimental.pallas import tpu_sc as plsc

_BATCH = 16384
_DIM = 64
_NCLS = 1000000
_NSAMP = 100
_NPAD = 128
_NW = 32
_LPW = _BATCH // _NW
_BPAD = 1000448
_BGRAN = _BPAD // 128

_TW = 4096
_TGRID = 123
_PROWS = _TGRID * _TW

_BLK = 2048
_GRID = _BATCH // _BLK


def _tr_body(slab_a_ref, slab_b_ref, eye_ref, out_ref):
    eye = eye_ref[...]
    sa = slab_a_ref[...].astype(jnp.bfloat16)
    sb = slab_b_ref[...].astype(jnp.bfloat16)
    ta = lax.dot_general(sa, eye, (((0,), (0,)), ((), ())),
                         preferred_element_type=jnp.float32)
    tb = lax.dot_general(sb, eye, (((0,), (0,)), ((), ())),
                         preferred_element_type=jnp.float32)
    out_ref[:, :_DIM] = ta
    out_ref[:, _DIM:] = tb


def _relayout(proj_t, eye):
    return pl.pallas_call(
        _tr_body,
        grid=(_TGRID,),
        in_specs=[
            pl.BlockSpec((_DIM, _TW), lambda k: (0, k)),
            pl.BlockSpec((_DIM, _TW),
                         lambda k: (0, jnp.minimum(k + _TGRID,
                                                   _NCLS // _TW))),
            pl.BlockSpec((_DIM, _DIM), lambda k: (0, 0)),
        ],
        out_specs=pl.BlockSpec((_TW, 128), lambda k: (k, 0)),
        out_shape=jax.ShapeDtypeStruct((_PROWS, 128), jnp.float32),
    )(proj_t, proj_t, eye)


def _sc_gather(proj2, bgran, ids3d, samp3d):
    mesh = plsc.VectorSubcoreMesh(core_axis_name="c", subcore_axis_name="s")
    cp = pltpu.CompilerParams()
    if "needs_layout_passes" in pltpu.CompilerParams.__dataclass_fields__:
        cp = dataclasses.replace(cp, needs_layout_passes=False)

    @functools.partial(
        pl.kernel,
        mesh=mesh,
        compiler_params=cp,
        out_type=(
            jax.ShapeDtypeStruct((_BATCH, 128), jnp.float32),
            jax.ShapeDtypeStruct((_NW, 1, _LPW), jnp.float32),
            jax.ShapeDtypeStruct((_NPAD, 128), jnp.float32),
            jax.ShapeDtypeStruct((1, _NPAD), jnp.float32),
        ),
        scratch_types=[
            pltpu.VMEM((1, _LPW), jnp.int32),
            pltpu.VMEM((4, 128), jnp.int32),
            pltpu.VMEM((4, 128), jnp.int32),
            pltpu.VMEM((_LPW, 128), jnp.float32),
            pltpu.VMEM((128, 128), jnp.float32),
            pltpu.VMEM((1, _LPW), jnp.float32),
            pltpu.VMEM((1, _NPAD), jnp.int32),
            pltpu.VMEM((_NPAD, 128), jnp.float32),
            pltpu.VMEM((1, _NPAD), jnp.float32),
            pltpu.SemaphoreType.DMA,
            pltpu.SemaphoreType.DMA,
        ],
    )
    def k(proj_hbm, bgran_hbm, ids_hbm, samp_hbm,
          rows_out, bias_out, ps_out, bs_out,
          ids_v, idxbuf, bidx, rowbuf, bgran_v, bl_v,
          samp_v, psbuf, bs_local, sem_g, sem_m):
        wid = lax.axis_index("s") * 2 + lax.axis_index("c")
        pltpu.sync_copy(ids_hbm.at[wid], ids_v)

        @pl.loop(0, _LPW // 16)
        def _mkidx(g):
            idv = ids_v[0, pl.ds(16 * g, 16)]
            row = g >> 3
            col = 16 * (g & 7)
            hi = (idv >= _PROWS).astype(jnp.int32)
            idxbuf[row, pl.ds(col, 16)] = idv - hi * _PROWS
            bidx[row, pl.ds(col, 16)] = idv >> 7

        for c in range(_LPW // 128):
            pltpu.async_copy(proj_hbm.at[idxbuf.at[c]],
                             rowbuf.at[pl.ds(128 * c, 128)], sem_g)

        for c in range(_LPW // 128):
            pltpu.sync_copy(bgran_hbm.at[bidx.at[c]], bgran_v)

            @pl.loop(0, 8)
            def _bx(g):
                j0 = 128 * c + 16 * g
                idv = ids_v[0, pl.ds(j0, 16)]
                rowv = 16 * g + lax.iota(jnp.int32, 16)
                bl_v[0, pl.ds(j0, 16)] = plsc.load_gather(
                    bgran_v, [rowv, idv & 127])

        for c in range(_LPW // 128):
            pltpu.make_async_copy(proj_hbm.at[idxbuf.at[c]],
                                  rowbuf.at[pl.ds(128 * c, 128)],
                                  sem_g).wait()

        pltpu.sync_copy(rowbuf,
                        rows_out.at[pl.ds(
                            pl.multiple_of(wid * _LPW, _LPW), _LPW)])
        pltpu.sync_copy(bl_v, bias_out.at[wid])

        @pl.when(wid == _NW - 1)
        def _samples():
            pltpu.sync_copy(samp_hbm.at[0], samp_v)

            @pl.loop(0, _NPAD // 16)
            def _mksidx(g):
                sv = samp_v[0, pl.ds(16 * g, 16)]
                shi = (sv >= _PROWS).astype(jnp.int32)
                idxbuf[0, pl.ds(16 * g, 16)] = sv - shi * _PROWS
                bidx[0, pl.ds(16 * g, 16)] = sv >> 7
            pltpu.async_copy(proj_hbm.at[idxbuf.at[0]], psbuf, sem_g)
            pltpu.sync_copy(bgran_hbm.at[bidx.at[0]], bgran_v)

            @pl.loop(0, _NPAD // 16)
            def _sx(g):
                sv = samp_v[0, pl.ds(16 * g, 16)]
                rowv = 16 * g + lax.iota(jnp.int32, 16)
                bs_local[0, pl.ds(16 * g, 16)] = plsc.load_gather(
                    bgran_v, [rowv, sv & 127])
            pltpu.make_async_copy(proj_hbm.at[idxbuf.at[0]], psbuf,
                                  sem_g).wait()
            pltpu.sync_copy(psbuf, ps_out)
            pltpu.sync_copy(bs_local, bs_out)

    return k(proj2, bgran, ids3d, samp3d)


def _tc_loss_body(pred_ref, rows_ref, bias_ref, lab_ref, ps_ref, bs_ref,
                  samp_ref, spar_ref, out_ref):
    i = pl.program_id(0)

    @pl.when(i == 0)
    def _init():
        out_ref[...] = jnp.zeros((1, 1), jnp.float32)

    pred = pred_ref[...]
    rows2 = rows_ref[...]
    bias_l = bias_ref[...]
    lab = lab_ref[...]
    ps2 = ps_ref[...]
    bs_row = bs_ref[...]
    samp = samp_ref[...]
    spar = spar_ref[...]

    odd = lab >= _PROWS
    ls_e = jnp.sum(pred * rows2[:, :_DIM], axis=1, keepdims=True)
    ls_o = jnp.sum(pred * rows2[:, _DIM:], axis=1, keepdims=True)
    ls = jnp.where(odd, ls_o, ls_e) + bias_l

    ps = jnp.where(spar == 1, ps2[:, _DIM:], ps2[:, :_DIM])
    noise = lax.dot_general(pred, ps, (((1,), (1,)), ((), ())),
                            preferred_element_type=jnp.float32)
    noise = noise + bs_row
    rej = (lab == samp)
    nrej = jnp.sum(rej.astype(jnp.float32), axis=1, keepdims=True)
    noise = noise - 1e6 * rej.astype(jnp.float32)
    noise = noise - jnp.log(float(_NSAMP) - nrej)
    col = lax.broadcasted_iota(jnp.int32, (_BLK, _NPAD), 1)
    noise = jnp.where(col < _NSAMP, noise, -1e30)

    m = jnp.maximum(ls, jnp.max(noise, axis=1, keepdims=True))
    se = jnp.exp(ls - m) + jnp.sum(jnp.exp(noise - m), axis=1, keepdims=True)
    nll = m + jnp.log(se) - ls
    out_ref[...] += jnp.sum(nll, axis=(0, 1), keepdims=True)

    @pl.when(i == _GRID - 1)
    def _fin():
        out_ref[...] = out_ref[...] / float(_BATCH)


def kernel(predictions, labels, projection, bias, samples):
    proj_t = jnp.transpose(projection)
    eye = jnp.eye(_DIM, dtype=jnp.bfloat16)
    proj2 = _relayout(proj_t, eye)

    ids3d = labels.reshape(_NW, 1, _LPW)
    samp_gather = jnp.concatenate(
        [samples, jnp.zeros((_NPAD - _NSAMP,), jnp.int32)]).reshape(1, 1, _NPAD)
    bias2d = jnp.pad(bias, (0, _BPAD - _NCLS)).reshape(_BGRAN, 128)

    rows, bias3, ps2, bs = _sc_gather(proj2, bias2d, ids3d, samp_gather)

    bs_row = bs + math.log(_NCLS - 1)
    samp2d = jnp.concatenate(
        [samples, jnp.full((_NPAD - _NSAMP,), -1, jnp.int32)]).reshape(1, _NPAD)
    spar = (jnp.concatenate(
        [samples, jnp.zeros((_NPAD - _NSAMP,), jnp.int32)])
        >= _PROWS).astype(jnp.int32).reshape(_NPAD, 1)
    lab2d = labels.reshape(_BATCH, 1)
    bias2 = bias3.reshape(_BATCH, 1)

    out = pl.pallas_call(
        _tc_loss_body,
        grid=(_GRID,),
        in_specs=[
            pl.BlockSpec((_BLK, _DIM), lambda i: (i, 0)),
            pl.BlockSpec((_BLK, 128), lambda i: (i, 0)),
            pl.BlockSpec((_BLK, 1), lambda i: (i, 0)),
            pl.BlockSpec((_BLK, 1), lambda i: (i, 0)),
            pl.BlockSpec((_NPAD, 128), lambda i: (0, 0)),
            pl.BlockSpec((1, _NPAD), lambda i: (0, 0)),
            pl.BlockSpec((1, _NPAD), lambda i: (0, 0)),
            pl.BlockSpec((_NPAD, 1), lambda i: (0, 0)),
        ],
        out_specs=pl.BlockSpec((1, 1), lambda i: (0, 0)),
        out_shape=jax.ShapeDtypeStruct((1, 1), jnp.float32),
    )(predictions, rows, bias2, lab2d, ps2, bs_row, samp2d, spar)
    return out[0, 0]


# speedup vs baseline: 3.2997x; 1.1349x over previous
import dataclasses
import functools
import math

import jax
import jax.numpy as jnp
from jax import lax
from jax.experimental import pallas as pl
from jax.experimental.pallas import tpu as pltpu
from jax.experimental.pallas import tpu_sc as plsc

_BATCH = 16384
_DIM = 64
_NCLS = 1000000
_NSAMP = 100
_NPAD = 128
_NW = 32
_LPW = _BATCH // _NW
_BPAD = 1000448
_BGRAN = _BPAD // 128

_TW = 8192
_TGRID = 62
_PROWS = _TGRID * _TW

_BLK = 2048
_GRID = _BATCH // _BLK


def _tr_body(slab_a_ref, slab_b_ref, eye_ref, out_ref):
    eye = eye_ref[...]
    sa = slab_a_ref[...].astype(jnp.bfloat16)
    sb = slab_b_ref[...].astype(jnp.bfloat16)
    ta = lax.dot_general(sa, eye, (((0,), (0,)), ((), ())),
                         preferred_element_type=jnp.float32)
    tb = lax.dot_general(sb, eye, (((0,), (0,)), ((), ())),
                         preferred_element_type=jnp.float32)
    out_ref[:, :_DIM] = ta
    out_ref[:, _DIM:] = tb


def _relayout(proj_t, eye):
    return pl.pallas_call(
        _tr_body,
        grid=(_TGRID,),
        in_specs=[
            pl.BlockSpec((_DIM, _TW), lambda k: (0, k)),
            pl.BlockSpec((_DIM, _TW),
                         lambda k: (0, jnp.minimum(k + _TGRID,
                                                   _NCLS // _TW))),
            pl.BlockSpec((_DIM, _DIM), lambda k: (0, 0)),
        ],
        out_specs=pl.BlockSpec((_TW, 128), lambda k: (k, 0)),
        out_shape=jax.ShapeDtypeStruct((_PROWS, 128), jnp.float32),
    )(proj_t, proj_t, eye)


def _sc_gather(proj2, bgran, ids3d, samp3d):
    mesh = plsc.VectorSubcoreMesh(core_axis_name="c", subcore_axis_name="s")
    cp = pltpu.CompilerParams()
    if "needs_layout_passes" in pltpu.CompilerParams.__dataclass_fields__:
        cp = dataclasses.replace(cp, needs_layout_passes=False)

    @functools.partial(
        pl.kernel,
        mesh=mesh,
        compiler_params=cp,
        out_type=(
            jax.ShapeDtypeStruct((_BATCH, 128), jnp.float32),
            jax.ShapeDtypeStruct((_NW, 1, _LPW), jnp.float32),
            jax.ShapeDtypeStruct((_NPAD, 128), jnp.float32),
            jax.ShapeDtypeStruct((1, _NPAD), jnp.float32),
        ),
        scratch_types=[
            pltpu.VMEM((1, _LPW), jnp.int32),
            pltpu.VMEM((4, 128), jnp.int32),
            pltpu.VMEM((4, 128), jnp.int32),
            pltpu.VMEM((_LPW, 128), jnp.float32),
            pltpu.VMEM((128, 128), jnp.float32),
            pltpu.VMEM((1, _LPW), jnp.float32),
            pltpu.VMEM((1, _NPAD), jnp.int32),
            pltpu.VMEM((_NPAD, 128), jnp.float32),
            pltpu.VMEM((1, _NPAD), jnp.float32),
            pltpu.SemaphoreType.DMA,
            pltpu.SemaphoreType.DMA,
        ],
    )
    def k(proj_hbm, bgran_hbm, ids_hbm, samp_hbm,
          rows_out, bias_out, ps_out, bs_out,
          ids_v, idxbuf, bidx, rowbuf, bgran_v, bl_v,
          samp_v, psbuf, bs_local, sem_g, sem_m):
        wid = lax.axis_index("s") * 2 + lax.axis_index("c")
        pltpu.sync_copy(ids_hbm.at[wid], ids_v)

        @pl.loop(0, _LPW // 16)
        def _mkidx(g):
            idv = ids_v[0, pl.ds(16 * g, 16)]
            row = g >> 3
            col = 16 * (g & 7)
            hi = (idv >= _PROWS).astype(jnp.int32)
            idxbuf[row, pl.ds(col, 16)] = idv - hi * _PROWS
            bidx[row, pl.ds(col, 16)] = idv >> 7

        for c in range(_LPW // 128):
            pltpu.async_copy(proj_hbm.at[idxbuf.at[c]],
                             rowbuf.at[pl.ds(128 * c, 128)], sem_g)

        for c in range(_LPW // 128):
            pltpu.sync_copy(bgran_hbm.at[bidx.at[c]], bgran_v)

            @pl.loop(0, 8)
            def _bx(g):
                j0 = 128 * c + 16 * g
                idv = ids_v[0, pl.ds(j0, 16)]
                rowv = 16 * g + lax.iota(jnp.int32, 16)
                bl_v[0, pl.ds(j0, 16)] = plsc.load_gather(
                    bgran_v, [rowv, idv & 127])

        for c in range(_LPW // 128):
            pltpu.make_async_copy(proj_hbm.at[idxbuf.at[c]],
                                  rowbuf.at[pl.ds(128 * c, 128)],
                                  sem_g).wait()

        pltpu.sync_copy(rowbuf,
                        rows_out.at[pl.ds(
                            pl.multiple_of(wid * _LPW, _LPW), _LPW)])
        pltpu.sync_copy(bl_v, bias_out.at[wid])

        @pl.when(wid == _NW - 1)
        def _samples():
            pltpu.sync_copy(samp_hbm.at[0], samp_v)

            @pl.loop(0, _NPAD // 16)
            def _mksidx(g):
                sv = samp_v[0, pl.ds(16 * g, 16)]
                shi = (sv >= _PROWS).astype(jnp.int32)
                idxbuf[0, pl.ds(16 * g, 16)] = sv - shi * _PROWS
                bidx[0, pl.ds(16 * g, 16)] = sv >> 7
            pltpu.async_copy(proj_hbm.at[idxbuf.at[0]], psbuf, sem_g)
            pltpu.sync_copy(bgran_hbm.at[bidx.at[0]], bgran_v)

            @pl.loop(0, _NPAD // 16)
            def _sx(g):
                sv = samp_v[0, pl.ds(16 * g, 16)]
                rowv = 16 * g + lax.iota(jnp.int32, 16)
                bs_local[0, pl.ds(16 * g, 16)] = plsc.load_gather(
                    bgran_v, [rowv, sv & 127])
            pltpu.make_async_copy(proj_hbm.at[idxbuf.at[0]], psbuf,
                                  sem_g).wait()
            pltpu.sync_copy(psbuf, ps_out)
            pltpu.sync_copy(bs_local, bs_out)

    return k(proj2, bgran, ids3d, samp3d)


def _tc_loss_body(pred_ref, rows_ref, bias_ref, lab_ref, ps_ref, bs_ref,
                  samp_ref, spar_ref, out_ref):
    i = pl.program_id(0)

    @pl.when(i == 0)
    def _init():
        out_ref[...] = jnp.zeros((1, 1), jnp.float32)

    pred = pred_ref[...]
    rows2 = rows_ref[...]
    bias_l = bias_ref[...]
    lab = lab_ref[...]
    ps2 = ps_ref[...]
    bs_row = bs_ref[...]
    samp = samp_ref[...]
    spar = spar_ref[...]

    odd = lab >= _PROWS
    ls_e = jnp.sum(pred * rows2[:, :_DIM], axis=1, keepdims=True)
    ls_o = jnp.sum(pred * rows2[:, _DIM:], axis=1, keepdims=True)
    ls = jnp.where(odd, ls_o, ls_e) + bias_l

    ps = jnp.where(spar == 1, ps2[:, _DIM:], ps2[:, :_DIM])
    noise = lax.dot_general(pred, ps, (((1,), (1,)), ((), ())),
                            preferred_element_type=jnp.float32)
    noise = noise + bs_row
    rej = (lab == samp)
    nrej = jnp.sum(rej.astype(jnp.float32), axis=1, keepdims=True)
    noise = noise - 1e6 * rej.astype(jnp.float32)
    noise = noise - jnp.log(float(_NSAMP) - nrej)
    col = lax.broadcasted_iota(jnp.int32, (_BLK, _NPAD), 1)
    noise = jnp.where(col < _NSAMP, noise, -1e30)

    m = jnp.maximum(ls, jnp.max(noise, axis=1, keepdims=True))
    se = jnp.exp(ls - m) + jnp.sum(jnp.exp(noise - m), axis=1, keepdims=True)
    nll = m + jnp.log(se) - ls
    out_ref[...] += jnp.sum(nll, axis=(0, 1), keepdims=True)

    @pl.when(i == _GRID - 1)
    def _fin():
        out_ref[...] = out_ref[...] / float(_BATCH)


def kernel(predictions, labels, projection, bias, samples):
    proj_t = jnp.transpose(projection)
    eye = jnp.eye(_DIM, dtype=jnp.bfloat16)
    proj2 = _relayout(proj_t, eye)

    ids3d = labels.reshape(_NW, 1, _LPW)
    samp_gather = jnp.concatenate(
        [samples, jnp.zeros((_NPAD - _NSAMP,), jnp.int32)]).reshape(1, 1, _NPAD)
    bias2d = jnp.pad(bias, (0, _BPAD - _NCLS)).reshape(_BGRAN, 128)

    rows, bias3, ps2, bs = _sc_gather(proj2, bias2d, ids3d, samp_gather)

    bs_row = bs + math.log(_NCLS - 1)
    samp2d = jnp.concatenate(
        [samples, jnp.full((_NPAD - _NSAMP,), -1, jnp.int32)]).reshape(1, _NPAD)
    spar = (jnp.concatenate(
        [samples, jnp.zeros((_NPAD - _NSAMP,), jnp.int32)])
        >= _PROWS).astype(jnp.int32).reshape(_NPAD, 1)
    lab2d = labels.reshape(_BATCH, 1)
    bias2 = bias3.reshape(_BATCH, 1)

    out = pl.pallas_call(
        _tc_loss_body,
        grid=(_GRID,),
        in_specs=[
            pl.BlockSpec((_BLK, _DIM), lambda i: (i, 0)),
            pl.BlockSpec((_BLK, 128), lambda i: (i, 0)),
            pl.BlockSpec((_BLK, 1), lambda i: (i, 0)),
            pl.BlockSpec((_BLK, 1), lambda i: (i, 0)),
            pl.BlockSpec((_NPAD, 128), lambda i: (0, 0)),
            pl.BlockSpec((1, _NPAD), lambda i: (0, 0)),
            pl.BlockSpec((1, _NPAD), lambda i: (0, 0)),
            pl.BlockSpec((_NPAD, 1), lambda i: (0, 0)),
        ],
        out_specs=pl.BlockSpec((1, 1), lambda i: (0, 0)),
        out_shape=jax.ShapeDtypeStruct((1, 1), jnp.float32),
    )(predictions, rows, bias2, lab2d, ps2, bs_row, samp2d, spar)
    return out[0, 0]


# speedup vs baseline: 3.5357x; 1.0715x over previous
import dataclasses
import functools
import math

import jax
import jax.numpy as jnp
from jax import lax
from jax.experimental import pallas as pl
from jax.experimental.pallas import tpu as pltpu
from jax.experimental.pallas import tpu_sc as plsc

_BATCH = 16384
_DIM = 64
_NCLS = 1000000
_NSAMP = 100
_NPAD = 128
_NW = 32
_LPW = _BATCH // _NW
_BPAD = 1000448
_BGRAN = _BPAD // 128

_TW = 16384
_TGRID = 31
_PROWS = _TGRID * _TW

_BLK = 2048
_GRID = _BATCH // _BLK


def _tr_body(slab_a_ref, slab_b_ref, eye_ref, out_ref):
    eye = eye_ref[...]
    sa = slab_a_ref[...].astype(jnp.bfloat16)
    sb = slab_b_ref[...].astype(jnp.bfloat16)
    ta = lax.dot_general(sa, eye, (((0,), (0,)), ((), ())),
                         preferred_element_type=jnp.float32)
    tb = lax.dot_general(sb, eye, (((0,), (0,)), ((), ())),
                         preferred_element_type=jnp.float32)
    out_ref[:, :_DIM] = ta
    out_ref[:, _DIM:] = tb


def _relayout(proj_t, eye):
    return pl.pallas_call(
        _tr_body,
        grid=(_TGRID,),
        in_specs=[
            pl.BlockSpec((_DIM, _TW), lambda k: (0, k)),
            pl.BlockSpec((_DIM, _TW),
                         lambda k: (0, jnp.minimum(k + _TGRID,
                                                   _NCLS // _TW))),
            pl.BlockSpec((_DIM, _DIM), lambda k: (0, 0)),
        ],
        out_specs=pl.BlockSpec((_TW, 128), lambda k: (k, 0)),
        out_shape=jax.ShapeDtypeStruct((_PROWS, 128), jnp.float32),
    )(proj_t, proj_t, eye)


def _sc_gather(proj2, bgran, ids3d, samp3d):
    mesh = plsc.VectorSubcoreMesh(core_axis_name="c", subcore_axis_name="s")
    cp = pltpu.CompilerParams()
    if "needs_layout_passes" in pltpu.CompilerParams.__dataclass_fields__:
        cp = dataclasses.replace(cp, needs_layout_passes=False)

    @functools.partial(
        pl.kernel,
        mesh=mesh,
        compiler_params=cp,
        out_type=(
            jax.ShapeDtypeStruct((_BATCH, 128), jnp.float32),
            jax.ShapeDtypeStruct((_NW, 1, _LPW), jnp.float32),
            jax.ShapeDtypeStruct((_NPAD, 128), jnp.float32),
            jax.ShapeDtypeStruct((1, _NPAD), jnp.float32),
        ),
        scratch_types=[
            pltpu.VMEM((1, _LPW), jnp.int32),
            pltpu.VMEM((4, 128), jnp.int32),
            pltpu.VMEM((4, 128), jnp.int32),
            pltpu.VMEM((_LPW, 128), jnp.float32),
            pltpu.VMEM((128, 128), jnp.float32),
            pltpu.VMEM((1, _LPW), jnp.float32),
            pltpu.VMEM((1, _NPAD), jnp.int32),
            pltpu.VMEM((_NPAD, 128), jnp.float32),
            pltpu.VMEM((1, _NPAD), jnp.float32),
            pltpu.SemaphoreType.DMA,
            pltpu.SemaphoreType.DMA,
        ],
    )
    def k(proj_hbm, bgran_hbm, ids_hbm, samp_hbm,
          rows_out, bias_out, ps_out, bs_out,
          ids_v, idxbuf, bidx, rowbuf, bgran_v, bl_v,
          samp_v, psbuf, bs_local, sem_g, sem_m):
        wid = lax.axis_index("s") * 2 + lax.axis_index("c")
        pltpu.sync_copy(ids_hbm.at[wid], ids_v)

        @pl.loop(0, _LPW // 16)
        def _mkidx(g):
            idv = ids_v[0, pl.ds(16 * g, 16)]
            row = g >> 3
            col = 16 * (g & 7)
            hi = (idv >= _PROWS).astype(jnp.int32)
            idxbuf[row, pl.ds(col, 16)] = idv - hi * _PROWS
            bidx[row, pl.ds(col, 16)] = idv >> 7

        for c in range(_LPW // 128):
            pltpu.async_copy(proj_hbm.at[idxbuf.at[c]],
                             rowbuf.at[pl.ds(128 * c, 128)], sem_g)

        for c in range(_LPW // 128):
            pltpu.sync_copy(bgran_hbm.at[bidx.at[c]], bgran_v)

            @pl.loop(0, 8)
            def _bx(g):
                j0 = 128 * c + 16 * g
                idv = ids_v[0, pl.ds(j0, 16)]
                rowv = 16 * g + lax.iota(jnp.int32, 16)
                bl_v[0, pl.ds(j0, 16)] = plsc.load_gather(
                    bgran_v, [rowv, idv & 127])

        for c in range(_LPW // 128):
            pltpu.make_async_copy(proj_hbm.at[idxbuf.at[c]],
                                  rowbuf.at[pl.ds(128 * c, 128)],
                                  sem_g).wait()

        pltpu.sync_copy(rowbuf,
                        rows_out.at[pl.ds(
                            pl.multiple_of(wid * _LPW, _LPW), _LPW)])
        pltpu.sync_copy(bl_v, bias_out.at[wid])

        @pl.when(wid == _NW - 1)
        def _samples():
            pltpu.sync_copy(samp_hbm.at[0], samp_v)

            @pl.loop(0, _NPAD // 16)
            def _mksidx(g):
                sv = samp_v[0, pl.ds(16 * g, 16)]
                shi = (sv >= _PROWS).astype(jnp.int32)
                idxbuf[0, pl.ds(16 * g, 16)] = sv - shi * _PROWS
                bidx[0, pl.ds(16 * g, 16)] = sv >> 7
            pltpu.async_copy(proj_hbm.at[idxbuf.at[0]], psbuf, sem_g)
            pltpu.sync_copy(bgran_hbm.at[bidx.at[0]], bgran_v)

            @pl.loop(0, _NPAD // 16)
            def _sx(g):
                sv = samp_v[0, pl.ds(16 * g, 16)]
                rowv = 16 * g + lax.iota(jnp.int32, 16)
                bs_local[0, pl.ds(16 * g, 16)] = plsc.load_gather(
                    bgran_v, [rowv, sv & 127])
            pltpu.make_async_copy(proj_hbm.at[idxbuf.at[0]], psbuf,
                                  sem_g).wait()
            pltpu.sync_copy(psbuf, ps_out)
            pltpu.sync_copy(bs_local, bs_out)

    return k(proj2, bgran, ids3d, samp3d)


def _tc_loss_body(pred_ref, rows_ref, bias_ref, lab_ref, ps_ref, bs_ref,
                  samp_ref, spar_ref, out_ref):
    i = pl.program_id(0)

    @pl.when(i == 0)
    def _init():
        out_ref[...] = jnp.zeros((1, 1), jnp.float32)

    pred = pred_ref[...]
    rows2 = rows_ref[...]
    bias_l = bias_ref[...]
    lab = lab_ref[...]
    ps2 = ps_ref[...]
    bs_row = bs_ref[...]
    samp = samp_ref[...]
    spar = spar_ref[...]

    odd = lab >= _PROWS
    ls_e = jnp.sum(pred * rows2[:, :_DIM], axis=1, keepdims=True)
    ls_o = jnp.sum(pred * rows2[:, _DIM:], axis=1, keepdims=True)
    ls = jnp.where(odd, ls_o, ls_e) + bias_l

    ps = jnp.where(spar == 1, ps2[:, _DIM:], ps2[:, :_DIM])
    noise = lax.dot_general(pred, ps, (((1,), (1,)), ((), ())),
                            preferred_element_type=jnp.float32)
    noise = noise + bs_row
    rej = (lab == samp)
    nrej = jnp.sum(rej.astype(jnp.float32), axis=1, keepdims=True)
    noise = noise - 1e6 * rej.astype(jnp.float32)
    noise = noise - jnp.log(float(_NSAMP) - nrej)
    col = lax.broadcasted_iota(jnp.int32, (_BLK, _NPAD), 1)
    noise = jnp.where(col < _NSAMP, noise, -1e30)

    m = jnp.maximum(ls, jnp.max(noise, axis=1, keepdims=True))
    se = jnp.exp(ls - m) + jnp.sum(jnp.exp(noise - m), axis=1, keepdims=True)
    nll = m + jnp.log(se) - ls
    out_ref[...] += jnp.sum(nll, axis=(0, 1), keepdims=True)

    @pl.when(i == _GRID - 1)
    def _fin():
        out_ref[...] = out_ref[...] / float(_BATCH)


def kernel(predictions, labels, projection, bias, samples):
    proj_t = jnp.transpose(projection)
    eye = jnp.eye(_DIM, dtype=jnp.bfloat16)
    proj2 = _relayout(proj_t, eye)

    ids3d = labels.reshape(_NW, 1, _LPW)
    samp_gather = jnp.concatenate(
        [samples, jnp.zeros((_NPAD - _NSAMP,), jnp.int32)]).reshape(1, 1, _NPAD)
    bias2d = jnp.pad(bias, (0, _BPAD - _NCLS)).reshape(_BGRAN, 128)

    rows, bias3, ps2, bs = _sc_gather(proj2, bias2d, ids3d, samp_gather)

    bs_row = bs + math.log(_NCLS - 1)
    samp2d = jnp.concatenate(
        [samples, jnp.full((_NPAD - _NSAMP,), -1, jnp.int32)]).reshape(1, _NPAD)
    spar = (jnp.concatenate(
        [samples, jnp.zeros((_NPAD - _NSAMP,), jnp.int32)])
        >= _PROWS).astype(jnp.int32).reshape(_NPAD, 1)
    lab2d = labels.reshape(_BATCH, 1)
    bias2 = bias3.reshape(_BATCH, 1)

    out = pl.pallas_call(
        _tc_loss_body,
        grid=(_GRID,),
        in_specs=[
            pl.BlockSpec((_BLK, _DIM), lambda i: (i, 0)),
            pl.BlockSpec((_BLK, 128), lambda i: (i, 0)),
            pl.BlockSpec((_BLK, 1), lambda i: (i, 0)),
            pl.BlockSpec((_BLK, 1), lambda i: (i, 0)),
            pl.BlockSpec((_NPAD, 128), lambda i: (0, 0)),
            pl.BlockSpec((1, _NPAD), lambda i: (0, 0)),
            pl.BlockSpec((1, _NPAD), lambda i: (0, 0)),
            pl.BlockSpec((_NPAD, 1), lambda i: (0, 0)),
        ],
        out_specs=pl.BlockSpec((1, 1), lambda i: (0, 0)),
        out_shape=jax.ShapeDtypeStruct((1, 1), jnp.float32),
    )(predictions, rows, bias2, lab2d, ps2, bs_row, samp2d, spar)
    return out[0, 0]
